# baseline (speedup 1.0000x reference)
import functools

import jax
import jax.numpy as jnp
from jax import lax
from jax.experimental import pallas as pl
from jax.experimental.pallas import tpu as pltpu
from jax.experimental.pallas import tpu_sc as plsc

N = 10000
E = 320000
IN = 128
H = 256
OUT = 128

NC = 2
NS = 16
NW = NC * NS
EPW = E // NW
NP = 10240
RPS = NP // NS
HR = NP // 16
HRS = HR // NS

_cache = {}


def _agg_kernel(with_count):
    if ("agg", with_count) in _cache:
        return _cache[("agg", with_count)]

    CHUNK, NH, NBUF = 50, 2, 4
    NCHUNK = EPW // CHUNK
    HC = NCHUNK // NH
    GPH = HC // NBUF

    mesh = plsc.VectorSubcoreMesh(core_axis_name="c", subcore_axis_name="s")

    out_type = [jax.ShapeDtypeStruct((NC, NP, 128), jnp.float32)]
    scratch = (
        [pltpu.VMEM((HC, CHUNK), jnp.int32),
         pltpu.VMEM((HC, CHUNK), jnp.int32)]
        + [pltpu.VMEM((CHUNK, 128), jnp.float32) for _ in range(NBUF)]
        + [pltpu.VMEM_SHARED((NP, 128), jnp.float32)]
        + [pltpu.SemaphoreType.DMA for _ in range(2 + 2 * NBUF)]
    )
    if with_count:
        out_type.append(jax.ShapeDtypeStruct((NC, HR, 16), jnp.float32))
        scratch += [
            pltpu.VMEM((HR, 16), jnp.float32),
            pltpu.VMEM((HR,), jnp.int32),
            pltpu.VMEM_SHARED((HR, 16), jnp.float32),
        ]

    @functools.partial(
        pl.kernel,
        out_type=out_type,
        mesh=mesh,
        scratch_types=scratch,
        compiler_params=pltpu.CompilerParams(
            use_tc_tiling_on_sc=False,
            needs_layout_passes=not with_count),
    )
    def agg(data_hbm, src_hbm, dst_hbm, zeros_hbm, *rest):
        if with_count:
            (out_hbm, outc_hbm, sidx, didx, *scr) = rest
            hist = scr[3 * NBUF + 3]
            iotar = scr[3 * NBUF + 4]
            cnt_acc = scr[3 * NBUF + 5]
        else:
            (out_hbm, sidx, didx, *scr) = rest
        rows = scr[:NBUF]
        acc = scr[NBUF]
        psem = scr[NBUF + 1]
        isem = scr[NBUF + 2]
        gsem = scr[NBUF + 3:2 * NBUF + 3]
        ssem = scr[2 * NBUF + 3:2 * NBUF + 3 + NBUF]
        cid = lax.axis_index("c")
        sid = lax.axis_index("s")
        wid = sid * NC + cid

        def gather(c, b):
            pltpu.async_copy(data_hbm.at[sidx.at[c]], rows[b], gsem[b])

        def scatter(c, b):
            pltpu.async_copy(rows[b], acc.at[didx.at[c]], ssem[b], add=True)

        def wait(sem, b):
            pltpu.make_async_copy(data_hbm.at[pl.ds(0, CHUNK)], rows[b], sem).wait()

        if with_count:
            ones16 = jnp.ones((16,), jnp.float32)
            nfull = CHUNK // 16
            tailmask = jnp.arange(16, dtype=jnp.int32) >= (16 * nfull - (CHUNK - 16))
            slices = [(off * 16, None) for off in range(nfull)]
            if CHUNK % 16:
                slices.append((CHUNK - 16, tailmask))

            def histo(c):
                for off, mask in slices:
                    v = didx[c, pl.ds(off, 16)]
                    hrow = lax.shift_right_logical(v, 4)
                    hcol = lax.bitwise_and(v, 15)
                    plsc.addupdate_scatter(hist, [hrow, hcol], ones16, mask=mask)

            zer16 = jnp.zeros((16,), jnp.float32)
            base16 = jnp.arange(16, dtype=jnp.int32)

            @pl.loop(0, HR)
            def _(rr):
                hist[rr] = zer16

            @pl.loop(0, HR // 16)
            def _(k):
                iotar[pl.ds(k * 16, 16)] = base16 + k * 16

            @pl.when(sid == 0)
            def _():
                pltpu.sync_copy(zeros_hbm.at[pl.ds(0, HR), pl.ds(0, 16)], cnt_acc)
        else:
            def histo(c):
                pass

        pltpu.async_copy(zeros_hbm, acc.at[pl.ds(sid * RPS, RPS)], psem)

        for h in range(NH):
            pltpu.async_copy(src_hbm.at[wid, h], sidx, isem)
            pltpu.async_copy(dst_hbm.at[wid, h], didx, isem)
            pltpu.make_async_copy(src_hbm.at[wid, h], sidx, isem).wait()
            pltpu.make_async_copy(dst_hbm.at[wid, h], didx, isem).wait()

            for b in range(NBUF):
                gather(b, b)

            if h == 0:
                pltpu.make_async_copy(
                    zeros_hbm, acc.at[pl.ds(sid * RPS, RPS)], psem).wait()
                plsc.subcore_barrier()

            @pl.loop(0, GPH - 1)
            def _(g):
                c = g * NBUF
                for b in range(NBUF):
                    wait(gsem[b], b)
                    scatter(c + b, b)
                    histo(c + b)
                for b in range(NBUF):
                    wait(ssem[b], b)
                    gather(c + NBUF + b, b)

            c = (GPH - 1) * NBUF
            for b in range(NBUF):
                wait(gsem[b], b)
                scatter(c + b, b)
                histo(c + b)
            for b in range(NBUF):
                wait(ssem[b], b)

        if with_count:
            pltpu.sync_copy(hist, cnt_acc.at[iotar], add=True)

        plsc.subcore_barrier()
        pltpu.sync_copy(acc.at[pl.ds(sid * RPS, RPS)],
                        out_hbm.at[cid, pl.ds(sid * RPS, RPS)])
        if with_count:
            pltpu.sync_copy(cnt_acc.at[pl.ds(sid * HRS, HRS)],
                            outc_hbm.at[cid, pl.ds(sid * HRS, HRS)])

    _cache[("agg", with_count)] = agg
    return agg


def _l1_body(agg_ref, cnt_ref, x_ref, w1l_ref, w1r_ref, b1_ref, w2l_ref,
             w2r_ref, p_ref, r_ref, ic_ref):
    a = agg_ref[0] + agg_ref[1]
    cnt = jnp.sum(cnt_ref[...], axis=1, keepdims=True)
    inv = 1.0 / jnp.maximum(cnt, 1.0)
    mean = a * inv
    h = jnp.dot(mean, w1l_ref[...], preferred_element_type=jnp.float32)
    h += jnp.dot(x_ref[...], w1r_ref[...], preferred_element_type=jnp.float32)
    h = jnp.maximum(h + b1_ref[...], 0.0)
    p_ref[...] = jnp.dot(h, w2l_ref[...], preferred_element_type=jnp.float32)
    r_ref[...] = jnp.dot(h, w2r_ref[...], preferred_element_type=jnp.float32)
    ic_ref[...] = jnp.broadcast_to(inv, ic_ref.shape)


def _l2_body(agg_ref, r_ref, ic_ref, b2_ref, o_ref):
    a = agg_ref[0] + agg_ref[1]
    z = a * ic_ref[:, 0:1] + r_ref[...] + b2_ref[...]
    m = jnp.max(z, axis=1, keepdims=True)
    z = z - m
    o_ref[...] = z - jnp.log(jnp.sum(jnp.exp(z), axis=1, keepdims=True))


def _layer1(agg1, cnt_t, x, w1l, w1r, b1, w2l, w2r):
    R = 1000
    full = lambda i: (0, 0)
    return pl.pallas_call(
        _l1_body,
        grid=(N // R,),
        in_specs=[
            pl.BlockSpec((NC, R, IN), lambda i: (0, i, 0)),
            pl.BlockSpec((R, NC), lambda i: (i, 0)),
            pl.BlockSpec((R, IN), lambda i: (i, 0)),
            pl.BlockSpec((IN, H), full),
            pl.BlockSpec((IN, H), full),
            pl.BlockSpec((1, H), full),
            pl.BlockSpec((H, OUT), full),
            pl.BlockSpec((H, OUT), full),
        ],
        out_specs=[
            pl.BlockSpec((R, OUT), lambda i: (i, 0)),
            pl.BlockSpec((R, OUT), lambda i: (i, 0)),
            pl.BlockSpec((R, 16), lambda i: (i, 0)),
        ],
        out_shape=[
            jax.ShapeDtypeStruct((N, OUT), jnp.float32),
            jax.ShapeDtypeStruct((N, OUT), jnp.float32),
            jax.ShapeDtypeStruct((N, 16), jnp.float32),
        ],
    )(agg1, cnt_t, x, w1l, w1r, b1, w2l, w2r)


def _layer2(agg2, r, ic, b2):
    R = 1000
    return pl.pallas_call(
        _l2_body,
        grid=(N // R,),
        in_specs=[
            pl.BlockSpec((NC, R, OUT), lambda i: (0, i, 0)),
            pl.BlockSpec((R, OUT), lambda i: (i, 0)),
            pl.BlockSpec((R, 16), lambda i: (i, 0)),
            pl.BlockSpec((1, OUT), lambda i: (0, 0)),
        ],
        out_specs=pl.BlockSpec((R, OUT), lambda i: (i, 0)),
        out_shape=jax.ShapeDtypeStruct((N, OUT), jnp.float32),
    )(agg2, r, ic, b2)


def _edge_views(edge_index, chunk, nh):
    hc = EPW // chunk // nh
    src = edge_index[0].reshape(NW, nh, hc, chunk)
    dst = edge_index[1].reshape(NW, nh, hc, chunk)
    return src, dst


def kernel(x, edge_index, W1l, W1r, b1, W2l, W2r, b2):
    zeros = jnp.zeros((RPS, 128), jnp.float32)
    src1, dst1 = _edge_views(edge_index, 50, 2)
    agg1, cnt = _agg_kernel(True)(x, src1, dst1, zeros)
    cnt_t = cnt.reshape(NC, NP).T
    p, r, ic = _layer1(agg1, cnt_t, x, W1l, W1r, b1.reshape(1, H), W2l, W2r)
    src2, dst2 = _edge_views(edge_index, 50, 2)
    agg2, = _agg_kernel(False)(p, src2, dst2, zeros)
    return _layer2(agg2, r, ic, b2.reshape(1, OUT))


# speedup vs baseline: 1.0293x; 1.0293x over previous
import functools

import jax
import jax.numpy as jnp
from jax import lax
from jax.experimental import pallas as pl
from jax.experimental.pallas import tpu as pltpu
from jax.experimental.pallas import tpu_sc as plsc

N = 10000
E = 320000
IN = 128
H = 256
OUT = 128

NC = 2
NS = 16
NW = NC * NS
EPW = E // NW
NP = 10240
RPS = NP // NS
HR = NP // 16
HRS = HR // NS

_cache = {}


def _agg_kernel(with_count):
    if ("agg", with_count) in _cache:
        return _cache[("agg", with_count)]

    if with_count:
        CHUNK, NH, NBUF = 40, 5, 5
    else:
        CHUNK, NH, NBUF = 50, 2, 4
    NCHUNK = EPW // CHUNK
    HC = NCHUNK // NH
    GPH = HC // NBUF

    mesh = plsc.VectorSubcoreMesh(core_axis_name="c", subcore_axis_name="s")

    out_type = [jax.ShapeDtypeStruct((NC, NP, 128), jnp.float32)]
    scratch = (
        [pltpu.VMEM((HC, CHUNK), jnp.int32),
         pltpu.VMEM((HC, CHUNK), jnp.int32)]
        + [pltpu.VMEM((CHUNK, 128), jnp.float32) for _ in range(NBUF)]
        + [pltpu.VMEM_SHARED((NP, 128), jnp.float32)]
        + [pltpu.SemaphoreType.DMA for _ in range(2 + 2 * NBUF)]
    )
    if with_count:
        out_type.append(jax.ShapeDtypeStruct((NC, HR, 16), jnp.float32))
        scratch += [
            pltpu.VMEM((HR, 16), jnp.float32),
            pltpu.VMEM((HR,), jnp.int32),
            pltpu.VMEM_SHARED((HR, 16), jnp.float32),
        ]

    @functools.partial(
        pl.kernel,
        out_type=out_type,
        mesh=mesh,
        scratch_types=scratch,
        compiler_params=pltpu.CompilerParams(
            use_tc_tiling_on_sc=False,
            needs_layout_passes=not with_count),
    )
    def agg(data_hbm, src_hbm, dst_hbm, zeros_hbm, *rest):
        if with_count:
            (out_hbm, outc_hbm, sidx, didx, *scr) = rest
            hist = scr[3 * NBUF + 3]
            iotar = scr[3 * NBUF + 4]
            cnt_acc = scr[3 * NBUF + 5]
        else:
            (out_hbm, sidx, didx, *scr) = rest
        rows = scr[:NBUF]
        acc = scr[NBUF]
        psem = scr[NBUF + 1]
        isem = scr[NBUF + 2]
        gsem = scr[NBUF + 3:2 * NBUF + 3]
        ssem = scr[2 * NBUF + 3:2 * NBUF + 3 + NBUF]
        cid = lax.axis_index("c")
        sid = lax.axis_index("s")
        wid = sid * NC + cid

        def gather(c, b):
            pltpu.async_copy(data_hbm.at[sidx.at[c]], rows[b], gsem[b])

        def scatter(c, b):
            pltpu.async_copy(rows[b], acc.at[didx.at[c]], ssem[b], add=True)

        def wait(sem, b):
            pltpu.make_async_copy(data_hbm.at[pl.ds(0, CHUNK)], rows[b], sem).wait()

        if with_count:
            ones16 = jnp.ones((16,), jnp.float32)
            nfull = CHUNK // 16
            tailmask = jnp.arange(16, dtype=jnp.int32) >= (16 * nfull - (CHUNK - 16))
            slices = [(off * 16, None) for off in range(nfull)]
            if CHUNK % 16:
                slices.append((CHUNK - 16, tailmask))

            def histo(c):
                for off, mask in slices:
                    v = didx[c, pl.ds(off, 16)]
                    hrow = lax.shift_right_logical(v, 4)
                    hcol = lax.bitwise_and(v, 15)
                    plsc.addupdate_scatter(hist, [hrow, hcol], ones16, mask=mask)

            zer16 = jnp.zeros((16,), jnp.float32)
            base16 = jnp.arange(16, dtype=jnp.int32)

            @pl.loop(0, HR)
            def _(rr):
                hist[rr] = zer16

            @pl.loop(0, HR // 16)
            def _(k):
                iotar[pl.ds(k * 16, 16)] = base16 + k * 16

            @pl.when(sid == 0)
            def _():
                pltpu.sync_copy(zeros_hbm.at[pl.ds(0, HR), pl.ds(0, 16)], cnt_acc)
        else:
            def histo(c):
                pass

        pltpu.async_copy(zeros_hbm, acc.at[pl.ds(sid * RPS, RPS)], psem)

        for h in range(NH):
            pltpu.async_copy(src_hbm.at[wid, h], sidx, isem)
            pltpu.async_copy(dst_hbm.at[wid, h], didx, isem)
            pltpu.make_async_copy(src_hbm.at[wid, h], sidx, isem).wait()
            pltpu.make_async_copy(dst_hbm.at[wid, h], didx, isem).wait()

            for b in range(NBUF):
                gather(b, b)

            if h == 0:
                pltpu.make_async_copy(
                    zeros_hbm, acc.at[pl.ds(sid * RPS, RPS)], psem).wait()
                plsc.subcore_barrier()

            @pl.loop(0, GPH - 1)
            def _(g):
                c = g * NBUF
                for b in range(NBUF):
                    wait(gsem[b], b)
                    scatter(c + b, b)
                    histo(c + b)
                for b in range(NBUF):
                    wait(ssem[b], b)
                    gather(c + NBUF + b, b)

            c = (GPH - 1) * NBUF
            for b in range(NBUF):
                wait(gsem[b], b)
                scatter(c + b, b)
                histo(c + b)
            for b in range(NBUF):
                wait(ssem[b], b)

        if with_count:
            pltpu.sync_copy(hist, cnt_acc.at[iotar], add=True)

        plsc.subcore_barrier()
        pltpu.sync_copy(acc.at[pl.ds(sid * RPS, RPS)],
                        out_hbm.at[cid, pl.ds(sid * RPS, RPS)])
        if with_count:
            pltpu.sync_copy(cnt_acc.at[pl.ds(sid * HRS, HRS)],
                            outc_hbm.at[cid, pl.ds(sid * HRS, HRS)])

    _cache[("agg", with_count)] = agg
    return agg


def _l1_body(agg_ref, cnt_ref, x_ref, w1l_ref, w1r_ref, b1_ref, w2l_ref,
             w2r_ref, p_ref, r_ref, ic_ref):
    a = agg_ref[0] + agg_ref[1]
    cnt = jnp.sum(cnt_ref[...], axis=1, keepdims=True)
    inv = 1.0 / jnp.maximum(cnt, 1.0)
    mean = a * inv
    h = jnp.dot(mean, w1l_ref[...], preferred_element_type=jnp.float32)
    h += jnp.dot(x_ref[...], w1r_ref[...], preferred_element_type=jnp.float32)
    h = jnp.maximum(h + b1_ref[...], 0.0)
    p_ref[...] = jnp.dot(h, w2l_ref[...], preferred_element_type=jnp.float32)
    r_ref[...] = jnp.dot(h, w2r_ref[...], preferred_element_type=jnp.float32)
    ic_ref[...] = jnp.broadcast_to(inv, ic_ref.shape)


def _l2_body(agg_ref, r_ref, ic_ref, b2_ref, o_ref):
    a = agg_ref[0] + agg_ref[1]
    z = a * ic_ref[:, 0:1] + r_ref[...] + b2_ref[...]
    m = jnp.max(z, axis=1, keepdims=True)
    z = z - m
    o_ref[...] = z - jnp.log(jnp.sum(jnp.exp(z), axis=1, keepdims=True))


def _layer1(agg1, cnt_t, x, w1l, w1r, b1, w2l, w2r):
    R = 1000
    full = lambda i: (0, 0)
    return pl.pallas_call(
        _l1_body,
        grid=(N // R,),
        in_specs=[
            pl.BlockSpec((NC, R, IN), lambda i: (0, i, 0)),
            pl.BlockSpec((R, NC), lambda i: (i, 0)),
            pl.BlockSpec((R, IN), lambda i: (i, 0)),
            pl.BlockSpec((IN, H), full),
            pl.BlockSpec((IN, H), full),
            pl.BlockSpec((1, H), full),
            pl.BlockSpec((H, OUT), full),
            pl.BlockSpec((H, OUT), full),
        ],
        out_specs=[
            pl.BlockSpec((R, OUT), lambda i: (i, 0)),
            pl.BlockSpec((R, OUT), lambda i: (i, 0)),
            pl.BlockSpec((R, 16), lambda i: (i, 0)),
        ],
        out_shape=[
            jax.ShapeDtypeStruct((N, OUT), jnp.float32),
            jax.ShapeDtypeStruct((N, OUT), jnp.float32),
            jax.ShapeDtypeStruct((N, 16), jnp.float32),
        ],
    )(agg1, cnt_t, x, w1l, w1r, b1, w2l, w2r)


def _layer2(agg2, r, ic, b2):
    R = 1000
    return pl.pallas_call(
        _l2_body,
        grid=(N // R,),
        in_specs=[
            pl.BlockSpec((NC, R, OUT), lambda i: (0, i, 0)),
            pl.BlockSpec((R, OUT), lambda i: (i, 0)),
            pl.BlockSpec((R, 16), lambda i: (i, 0)),
            pl.BlockSpec((1, OUT), lambda i: (0, 0)),
        ],
        out_specs=pl.BlockSpec((R, OUT), lambda i: (i, 0)),
        out_shape=jax.ShapeDtypeStruct((N, OUT), jnp.float32),
    )(agg2, r, ic, b2)


def _edge_views(edge_index, chunk, nh):
    hc = EPW // chunk // nh
    src = edge_index[0].reshape(NW, nh, hc, chunk)
    dst = edge_index[1].reshape(NW, nh, hc, chunk)
    return src, dst


def kernel(x, edge_index, W1l, W1r, b1, W2l, W2r, b2):
    zeros = jnp.zeros((RPS, 128), jnp.float32)
    src1, dst1 = _edge_views(edge_index, 40, 5)
    agg1, cnt = _agg_kernel(True)(x, src1, dst1, zeros)
    cnt_t = cnt.reshape(NC, NP).T
    p, r, ic = _layer1(agg1, cnt_t, x, W1l, W1r, b1.reshape(1, H), W2l, W2r)
    src2, dst2 = _edge_views(edge_index, 50, 2)
    agg2, = _agg_kernel(False)(p, src2, dst2, zeros)
    return _layer2(agg2, r, ic, b2.reshape(1, OUT))
